# Initial kernel scaffold; baseline (speedup 1.0000x reference)
import jax
import jax.numpy as jnp
from jax.experimental import pallas as pl


def kernel(x, codebooks):
    raise NotImplementedError("write your pallas kernel here")


# speedup vs baseline: 1.2162x; 1.2162x over previous
import functools

import jax
import jax.numpy as jnp
from jax.experimental import pallas as pl
from jax.experimental.pallas import tpu as pltpu

_NUM_HEADS = 12
_CODEBOOK = 1024
_NUM_RES = 4
_HEAD_DIM = 64


def _hvq_body(xt_ref, cb_ref, out_ref, idx_ref, perp_ref, counts_ref, *, n_tok, n_batch):
    h = pl.program_id(0)
    b = pl.program_id(1)

    @pl.when((h == 0) & (b == 0))
    def _init():
        counts_ref[...] = jnp.zeros_like(counts_ref)

    cbh = cb_ref[0]
    norm = jnp.sqrt(jnp.sum(cbh * cbh, axis=1, keepdims=True))
    cn = cbh / jnp.maximum(norm, 1e-12)

    resid = xt_ref[0, 0]
    acc = jnp.zeros_like(resid)
    for r in range(_NUM_RES):
        qn_nrm = jnp.sqrt(jnp.sum(resid * resid, axis=1, keepdims=True))
        qn = resid / jnp.maximum(qn_nrm, 1e-12)
        sim = jax.lax.dot_general(
            qn, cn, (((1,), (1,)), ((), ())),
            preferred_element_type=jnp.float32)
        idx = jnp.argmax(sim, axis=1).astype(jnp.int32)
        onehot = (jax.lax.broadcasted_iota(jnp.int32, sim.shape, 1)
                  == idx[:, None]).astype(jnp.float32)
        quant = jnp.dot(onehot, cbh, preferred_element_type=jnp.float32)
        acc = acc + quant
        resid = resid - quant
        idx_ref[0, 0, r, :] = idx
        counts_ref[h, r, :] = counts_ref[h, r, :] + jnp.sum(onehot, axis=0)
    out_ref[0, 0] = acc

    @pl.when((h == pl.num_programs(0) - 1) & (b == pl.num_programs(1) - 1))
    def _fin():
        mean = counts_ref[...] / float(n_batch * n_tok)
        ent = jnp.sum(mean * jnp.log(mean + 1e-10), axis=-1)
        perp_ref[...] = jnp.exp(-ent)


@jax.jit
def kernel(x, codebooks):
    bsz, n_tok, feat = x.shape
    h, m, d = codebooks.shape
    xt = x.reshape(bsz, n_tok, h, d).transpose(0, 2, 1, 3)

    grid = (h, bsz)
    out_q, idx_out, perp_out = pl.pallas_call(
        functools.partial(_hvq_body, n_tok=n_tok, n_batch=bsz),
        grid=grid,
        in_specs=[
            pl.BlockSpec((1, 1, n_tok, d), lambda hh, bb: (bb, hh, 0, 0)),
            pl.BlockSpec((1, m, d), lambda hh, bb: (hh, 0, 0)),
        ],
        out_specs=[
            pl.BlockSpec((1, 1, n_tok, d), lambda hh, bb: (bb, hh, 0, 0)),
            pl.BlockSpec((1, 1, _NUM_RES, n_tok), lambda hh, bb: (bb, hh, 0, 0)),
            pl.BlockSpec((h, _NUM_RES), lambda hh, bb: (0, 0)),
        ],
        out_shape=[
            jax.ShapeDtypeStruct((bsz, h, n_tok, d), jnp.float32),
            jax.ShapeDtypeStruct((bsz, h, _NUM_RES, n_tok), jnp.int32),
            jax.ShapeDtypeStruct((h, _NUM_RES), jnp.float32),
        ],
        scratch_shapes=[pltpu.VMEM((h, _NUM_RES, m), jnp.float32)],
    )(xt, codebooks)

    out = out_q.transpose(0, 2, 1, 3).reshape(bsz, n_tok, feat)
    indices = idx_out.transpose(0, 1, 3, 2).reshape(bsz, h, n_tok * _NUM_RES)
    perplexity = perp_out.reshape(h * _NUM_RES)
    return out, indices, perplexity


# speedup vs baseline: 1.2323x; 1.0133x over previous
import functools

import jax
import jax.numpy as jnp
from jax.experimental import pallas as pl
from jax.experimental.pallas import tpu as pltpu

_NUM_HEADS = 12
_CODEBOOK = 1024
_NUM_RES = 4
_HEAD_DIM = 64


def _hvq_body(xt_ref, cb_ref, out_ref, idx_ref, perp_ref, cn_ref, counts_ref,
              *, n_tok, n_batch):
    h = pl.program_id(0)
    b = pl.program_id(1)

    @pl.when((h == 0) & (b == 0))
    def _init():
        counts_ref[...] = jnp.zeros_like(counts_ref)

    cbh = cb_ref[0]

    @pl.when(b == 0)
    def _prep():
        nrm = jnp.sqrt(jnp.sum(cbh * cbh, axis=1, keepdims=True))
        cn_ref[...] = (cbh / jnp.maximum(nrm, 1e-12)).astype(jnp.bfloat16)

    cn = cn_ref[...]
    resid = xt_ref[0, 0]
    acc = jnp.zeros_like(resid)
    iota = jax.lax.broadcasted_iota(jnp.int32, (n_tok, _CODEBOOK), 1)
    ones_row = jnp.ones((1, n_tok), jnp.float32)
    for r in range(_NUM_RES):
        qn_nrm = jnp.sqrt(jnp.sum(resid * resid, axis=1, keepdims=True))
        qn = (resid / jnp.maximum(qn_nrm, 1e-12)).astype(jnp.bfloat16)
        sim = jax.lax.dot_general(
            qn, cn, (((1,), (1,)), ((), ())),
            preferred_element_type=jnp.float32)
        mx = jnp.max(sim, axis=1, keepdims=True)
        idx = jnp.min(jnp.where(sim == mx, iota, _CODEBOOK),
                      axis=1, keepdims=True)
        onehot = (iota == idx).astype(jnp.float32)
        quant = jnp.dot(onehot, cbh, preferred_element_type=jnp.float32)
        acc = acc + quant
        resid = resid - quant
        idx_ref[0, 0, :, r] = idx[:, 0]
        cnt = jnp.dot(ones_row, onehot, preferred_element_type=jnp.float32)
        counts_ref[h, r, :] = counts_ref[h, r, :] + cnt[0]
    out_ref[0, 0] = acc

    @pl.when((h == pl.num_programs(0) - 1) & (b == pl.num_programs(1) - 1))
    def _fin():
        mean = counts_ref[...] / float(n_batch * n_tok)
        ent = jnp.sum(mean * jnp.log(mean + 1e-10), axis=-1)
        perp_ref[...] = jnp.exp(-ent)


@jax.jit
def kernel(x, codebooks):
    bsz, n_tok, feat = x.shape
    h, m, d = codebooks.shape
    xt = x.reshape(bsz, n_tok, h, d).transpose(0, 2, 1, 3)

    grid = (h, bsz)
    out_q, idx_out, perp_out = pl.pallas_call(
        functools.partial(_hvq_body, n_tok=n_tok, n_batch=bsz),
        grid=grid,
        in_specs=[
            pl.BlockSpec((1, 1, n_tok, d), lambda hh, bb: (bb, hh, 0, 0)),
            pl.BlockSpec((1, m, d), lambda hh, bb: (hh, 0, 0)),
        ],
        out_specs=[
            pl.BlockSpec((1, 1, n_tok, d), lambda hh, bb: (bb, hh, 0, 0)),
            pl.BlockSpec((1, 1, n_tok, _NUM_RES), lambda hh, bb: (bb, hh, 0, 0)),
            pl.BlockSpec((h, _NUM_RES), lambda hh, bb: (0, 0)),
        ],
        out_shape=[
            jax.ShapeDtypeStruct((bsz, h, n_tok, d), jnp.float32),
            jax.ShapeDtypeStruct((bsz, h, n_tok, _NUM_RES), jnp.int32),
            jax.ShapeDtypeStruct((h, _NUM_RES), jnp.float32),
        ],
        scratch_shapes=[
            pltpu.VMEM((m, d), jnp.bfloat16),
            pltpu.VMEM((h, _NUM_RES, m), jnp.float32),
        ],
    )(xt, codebooks)

    out = out_q.transpose(0, 2, 1, 3).reshape(bsz, n_tok, feat)
    indices = idx_out.reshape(bsz, h, n_tok * _NUM_RES)
    perplexity = perp_out.reshape(h * _NUM_RES)
    return out, indices, perplexity


# speedup vs baseline: 1.8167x; 1.4742x over previous
import functools

import jax
import jax.numpy as jnp
from jax.experimental import pallas as pl
from jax.experimental.pallas import tpu as pltpu

_NUM_HEADS = 12
_CODEBOOK = 1024
_NUM_RES = 4
_HEAD_DIM = 64
_BGRP = 4


def _hvq_body(xt_ref, cb_ref, out_ref, idx_ref, perp_ref, cn_ref, cba_ref,
              counts_ref, *, n_tok, n_batch):
    h = pl.program_id(0)
    bg = pl.program_id(1)
    m_rows = _BGRP * n_tok

    @pl.when((h == 0) & (bg == 0))
    def _init():
        counts_ref[...] = jnp.zeros_like(counts_ref)

    @pl.when(bg == 0)
    def _prep():
        cbh = cb_ref[0]
        nrm = jnp.sqrt(jnp.sum(cbh * cbh, axis=1, keepdims=True))
        cn_ref[...] = (cbh / jnp.maximum(nrm, 1e-12)).astype(jnp.bfloat16)
        code = jax.lax.broadcasted_iota(jnp.int32, (_CODEBOOK, 1), 0)
        hi = (code // 16).astype(jnp.float32)
        lo = (code % 16).astype(jnp.float32)
        aug = jnp.concatenate(
            [cbh, hi, lo, jnp.zeros((_CODEBOOK, 62), jnp.float32)], axis=1)
        cba_ref[...] = aug.astype(jnp.bfloat16)

    cn = cn_ref[...]
    cba = cba_ref[...]
    resid = xt_ref[:, 0].reshape(m_rows, _HEAD_DIM)
    acc = jnp.zeros_like(resid)
    ones_row = jnp.ones((1, m_rows), jnp.float32)
    one_bf = jnp.ones((), jnp.bfloat16)
    zero_bf = jnp.zeros((), jnp.bfloat16)
    for r in range(_NUM_RES):
        qn_nrm = jnp.sqrt(jnp.sum(resid * resid, axis=1, keepdims=True))
        qn = (resid / jnp.maximum(qn_nrm, 1e-12)).astype(jnp.bfloat16)
        sim = jax.lax.dot_general(
            qn, cn, (((1,), (1,)), ((), ())),
            preferred_element_type=jnp.float32)
        mx = jnp.max(sim, axis=1, keepdims=True)
        onehot = jnp.where(sim == mx, 1.0, 0.0).astype(jnp.bfloat16)
        qa = jnp.dot(onehot, cba, preferred_element_type=jnp.float32)
        quant = qa[:, :_HEAD_DIM]
        idx = (qa[:, _HEAD_DIM] * 16.0 + qa[:, _HEAD_DIM + 1]).astype(jnp.int32)
        acc = acc + quant
        resid = resid - quant
        idx_ref[:, 0, :, r] = idx.reshape(_BGRP, n_tok)
        cnt = jax.lax.dot_general(
            ones_row, onehot, (((1,), (0,)), ((), ())),
            preferred_element_type=jnp.float32)
        counts_ref[h, r, :] = counts_ref[h, r, :] + cnt[0]
    out_ref[:, 0] = acc.reshape(_BGRP, n_tok, _HEAD_DIM)

    @pl.when((h == pl.num_programs(0) - 1) & (bg == pl.num_programs(1) - 1))
    def _fin():
        mean = counts_ref[...] / float(n_batch * n_tok)
        ent = jnp.sum(mean * jnp.log(mean + 1e-10), axis=-1)
        perp_ref[...] = jnp.exp(-ent)


@jax.jit
def kernel(x, codebooks):
    bsz, n_tok, feat = x.shape
    h, m, d = codebooks.shape
    xt = x.reshape(bsz, n_tok, h, d).transpose(0, 2, 1, 3)

    grid = (h, bsz // _BGRP)
    out_q, idx_out, perp_out = pl.pallas_call(
        functools.partial(_hvq_body, n_tok=n_tok, n_batch=bsz),
        grid=grid,
        in_specs=[
            pl.BlockSpec((_BGRP, 1, n_tok, d), lambda hh, bb: (bb, hh, 0, 0)),
            pl.BlockSpec((1, m, d), lambda hh, bb: (hh, 0, 0)),
        ],
        out_specs=[
            pl.BlockSpec((_BGRP, 1, n_tok, d), lambda hh, bb: (bb, hh, 0, 0)),
            pl.BlockSpec((_BGRP, 1, n_tok, _NUM_RES), lambda hh, bb: (bb, hh, 0, 0)),
            pl.BlockSpec((h, _NUM_RES), lambda hh, bb: (0, 0)),
        ],
        out_shape=[
            jax.ShapeDtypeStruct((bsz, h, n_tok, d), jnp.float32),
            jax.ShapeDtypeStruct((bsz, h, n_tok, _NUM_RES), jnp.int32),
            jax.ShapeDtypeStruct((h, _NUM_RES), jnp.float32),
        ],
        scratch_shapes=[
            pltpu.VMEM((m, d), jnp.bfloat16),
            pltpu.VMEM((m, 128), jnp.bfloat16),
            pltpu.VMEM((h, _NUM_RES, m), jnp.float32),
        ],
    )(xt, codebooks)

    out = out_q.transpose(0, 2, 1, 3).reshape(bsz, n_tok, feat)
    indices = idx_out.reshape(bsz, h, n_tok * _NUM_RES)
    perplexity = perp_out.reshape(h * _NUM_RES)
    return out, indices, perplexity
